# Initial kernel scaffold; baseline (speedup 1.0000x reference)
import jax
import jax.numpy as jnp
from jax.experimental import pallas as pl


def kernel(node_features, edge_index, W, b):
    raise NotImplementedError("write your pallas kernel here")


# speedup vs baseline: 18.7280x; 18.7280x over previous
import functools

import jax
import jax.numpy as jnp
from jax import lax
from jax.experimental import pallas as pl
from jax.experimental.pallas import tpu as pltpu
from jax.experimental.pallas import tpu_sc as plsc

N = 10000
E = 320000
D = 128
N_PAD = 10240

NC = 2
NS = 16
CHUNK = 128
N_CHUNKS = E // CHUNK
ROWS_PER_TILE = N_PAD // NS


def _mesh():
    return plsc.VectorSubcoreMesh(core_axis_name="c", subcore_axis_name="s")


def _hist_body(row_hbm, out_hbm, idx_v, ones_v, zero_v, hist, sem):
    cid = lax.axis_index("c")
    sid = lax.axis_index("s")
    wid = cid * NS + sid

    def zf(i, _):
        zero_v[pl.ds(i * 16, 16)] = jnp.zeros((16,), jnp.float32)
        return 0
    lax.fori_loop(0, ROWS_PER_TILE // 16, zf, 0)

    def of(i, _):
        ones_v[pl.ds(i * 16, 16)] = jnp.ones((16,), jnp.float32)
        return 0
    lax.fori_loop(0, CHUNK // 16, of, 0)

    pltpu.sync_copy(zero_v, hist.at[pl.ds(sid * ROWS_PER_TILE, ROWS_PER_TILE)])
    plsc.subcore_barrier()

    n_iters = (N_CHUNKS - wid + (NC * NS) - 1) // (NC * NS)

    def body(k, _):
        c = wid + k * (NC * NS)
        pltpu.sync_copy(row_hbm.at[pl.ds(c * CHUNK, CHUNK)], idx_v)
        pltpu.sync_copy(ones_v, hist.at[idx_v], add=True)
        return 0
    lax.fori_loop(0, n_iters, body, 0)

    plsc.subcore_barrier()
    pltpu.sync_copy(
        hist.at[pl.ds(sid * ROWS_PER_TILE, ROWS_PER_TILE)],
        out_hbm.at[cid, pl.ds(sid * ROWS_PER_TILE, ROWS_PER_TILE)],
    )


@jax.jit
def _histogram(row):
    return pl.kernel(
        _hist_body,
        out_type=jax.ShapeDtypeStruct((NC, N_PAD), jnp.float32),
        mesh=_mesh(),
        scratch_types=[
            pltpu.VMEM((CHUNK,), jnp.int32),
            pltpu.VMEM((CHUNK,), jnp.float32),
            pltpu.VMEM((ROWS_PER_TILE,), jnp.float32),
            pltpu.VMEM_SHARED((N_PAD,), jnp.float32),
            pltpu.SemaphoreType.DMA,
        ],
    )(row)


def _dense_block(x_ref, w_ref, b_ref, hp_ref, sup_ref, d_ref):
    deg = hp_ref[0] + hp_ref[1]
    d = lax.rsqrt(jnp.maximum(deg, 1.0))
    sup = lax.dot_general(
        x_ref[...], w_ref[...], (((1,), (1,)), ((), ())),
        preferred_element_type=jnp.float32,
    ) + b_ref[...]
    sup_ref[...] = sup * d
    d_ref[...] = d


@jax.jit
def _dense(x_pad, w, b2d, hist2):
    blk = 512
    grid = N_PAD // blk
    return pl.pallas_call(
        _dense_block,
        grid=(grid,),
        in_specs=[
            pl.BlockSpec((blk, D), lambda i: (i, 0)),
            pl.BlockSpec((D, D), lambda i: (0, 0)),
            pl.BlockSpec((1, D), lambda i: (0, 0)),
            pl.BlockSpec((NC, blk, 1), lambda i: (0, i, 0)),
        ],
        out_specs=[
            pl.BlockSpec((blk, D), lambda i: (i, 0)),
            pl.BlockSpec((blk, 1), lambda i: (i, 0)),
        ],
        out_shape=[
            jax.ShapeDtypeStruct((N_PAD, D), jnp.float32),
            jax.ShapeDtypeStruct((N_PAD, 1), jnp.float32),
        ],
    )(x_pad, w, b2d, hist2)


def _agg_body(sup_hbm, col_hbm, row_hbm, out_hbm, colv, rowv, rows_v, zbuf, acc, sem):
    cid = lax.axis_index("c")
    sid = lax.axis_index("s")
    wid = cid * NS + sid

    def zf(i, _):
        def zg(j, _):
            zbuf[i, pl.ds(j * 16, 16)] = jnp.zeros((16,), jnp.float32)
            return 0
        lax.fori_loop(0, D // 16, zg, 0)
        return 0
    lax.fori_loop(0, CHUNK, zf, 0)

    base = sid * ROWS_PER_TILE
    def zc(i, _):
        pltpu.sync_copy(zbuf, acc.at[pl.ds(base + i * CHUNK, CHUNK)])
        return 0
    lax.fori_loop(0, ROWS_PER_TILE // CHUNK, zc, 0)
    plsc.subcore_barrier()

    n_iters = (N_CHUNKS - wid + (NC * NS) - 1) // (NC * NS)

    def body(k, _):
        c = wid + k * (NC * NS)
        pltpu.sync_copy(col_hbm.at[pl.ds(c * CHUNK, CHUNK)], colv)
        pltpu.async_copy(sup_hbm.at[colv], rows_v, sem).wait()
        pltpu.sync_copy(row_hbm.at[pl.ds(c * CHUNK, CHUNK)], rowv)
        pltpu.sync_copy(rows_v, acc.at[rowv], add=True)
        return 0
    lax.fori_loop(0, n_iters, body, 0)

    plsc.subcore_barrier()
    def wb(i, _):
        pltpu.sync_copy(acc.at[pl.ds(base + i * CHUNK, CHUNK)], zbuf)
        pltpu.sync_copy(zbuf, out_hbm.at[cid, pl.ds(base + i * CHUNK, CHUNK)])
        return 0
    lax.fori_loop(0, ROWS_PER_TILE // CHUNK, wb, 0)


@jax.jit
def _aggregate(sup, col, row):
    return pl.kernel(
        _agg_body,
        out_type=jax.ShapeDtypeStruct((NC, N_PAD, D), jnp.float32),
        mesh=_mesh(),
        scratch_types=[
            pltpu.VMEM((CHUNK,), jnp.int32),
            pltpu.VMEM((CHUNK,), jnp.int32),
            pltpu.VMEM((CHUNK, D), jnp.float32),
            pltpu.VMEM((CHUNK, D), jnp.float32),
            pltpu.VMEM_SHARED((N_PAD, D), jnp.float32),
            pltpu.SemaphoreType.DMA,
        ],
    )(sup, col, row)


def _fin_block(p_ref, d_ref, o_ref):
    o_ref[...] = (p_ref[0] + p_ref[1]) * d_ref[...]


@jax.jit
def _finalize(partials, d):
    blk = 512
    grid = N_PAD // blk
    return pl.pallas_call(
        _fin_block,
        grid=(grid,),
        in_specs=[
            pl.BlockSpec((NC, blk, D), lambda i: (0, i, 0)),
            pl.BlockSpec((blk, 1), lambda i: (i, 0)),
        ],
        out_specs=pl.BlockSpec((blk, D), lambda i: (i, 0)),
        out_shape=jax.ShapeDtypeStruct((N_PAD, D), jnp.float32),
    )(partials, d)


def kernel(node_features, edge_index, W, b):
    row = edge_index[0]
    col = edge_index[1]
    x_pad = jnp.zeros((N_PAD, D), jnp.float32).at[:N].set(node_features)

    hist = _histogram(row)
    hist2 = hist[:, :, None]
    sup, d = _dense(x_pad, W, b.reshape(1, D), hist2)
    partials = _aggregate(sup, col, row)
    out = _finalize(partials, d)
    return out[:N]
